# baseline (speedup 1.0000x reference)
import jax
import jax.numpy as jnp
from jax import lax
from jax.experimental import pallas as pl
from jax.experimental.pallas import tpu as pltpu
from jax.experimental.pallas import tpu_sc as plsc

_B = 16384
_D = 128
_LOSS_WEIGHT = 0.005
_SCALE = 0.5 * _LOSS_WEIGHT / _B
_NC = 2
_NS = 16
_NW = _NC * _NS
_BPW = _B // _NW
_CHUNK = 128
_NBUF = 3
_CHUNKS = tuple((o, _CHUNK) for o in range(0, _BPW, _CHUNK))
_NCH = len(_CHUNKS)
_LANES = 16
_DBLK = _D // _LANES


def _center_loss_body(feat_hbm, y_hbm, w_hbm, gamma_hbm, out_hbm,
                      idx_v, gam_v, feat_v, w_v, acc_v,
                      gsem, fsem0, wsem0, wsem1, wsem2):
    wid = lax.axis_index("s") * _NC + lax.axis_index("c")
    base = wid * _BPW

    wsems = (wsem0, wsem1, wsem2)
    wcp = [None] * _NCH

    def fire(ch):
        buf = ch % _NBUF
        off, size = _CHUNKS[ch]
        wcp[ch] = pltpu.async_copy(
            w_hbm.at[idx_v.at[pl.ds(off, size)]],
            w_v.at[buf, pl.ds(0, size)], wsems[buf])

    fcp = pltpu.async_copy(
        feat_hbm.at[pl.ds(base, _BPW)], feat_v, fsem0)
    pltpu.sync_copy(y_hbm.at[pl.ds(base, _BPW)], idx_v)
    fire(0)
    gcopies = [
        pltpu.async_copy(
            gamma_hbm.at[idx_v.at[pl.ds(off, size)]],
            gam_v.at[pl.ds(off, size)],
            gsem)
        for off, size in _CHUNKS]
    fire(1)
    for cp in gcopies:
        cp.wait()

    acc = jnp.zeros((_LANES,), jnp.float32)
    for ch in range(_NCH):
        if ch + 2 < _NCH:
            fire(ch + 2)
        if ch == 0:
            fcp.wait()
        wcp[ch].wait()
        buf = ch % _NBUF
        off, size = _CHUNKS[ch]

        @plsc.parallel_loop(0, size, unroll=2, carry=acc)
        def acc(r, acc, off=off, buf=buf):
            l = jnp.bitwise_and(r, _LANES - 1)
            g16 = gam_v[pl.ds(off + (r - l), _LANES)]
            g = g16.at[jnp.full((_LANES,), l, jnp.int32)].get(
                mode="promise_in_bounds")
            for j in range(_DBLK):
                xv = feat_v[off + r, pl.ds(j * _LANES, _LANES)]
                wv = w_v[buf, r, pl.ds(j * _LANES, _LANES)]
                d = xv - g * wv
                acc = acc + d * d
            return acc

    acc_v[...] = acc * _SCALE
    pltpu.sync_copy(acc_v, out_hbm.at[wid])


@jax.jit
def _run(feat, y, w, gamma):
    mesh = plsc.VectorSubcoreMesh(core_axis_name="c", subcore_axis_name="s")
    out = pl.kernel(
        _center_loss_body,
        mesh=mesh,
        out_type=jax.ShapeDtypeStruct((_NW, _LANES), jnp.float32),
        scratch_types=[
            pltpu.VMEM((_BPW,), jnp.int32),
            pltpu.VMEM((_BPW,), jnp.float32),
            pltpu.VMEM((_BPW, _D), jnp.float32),
            pltpu.VMEM((_NBUF, _CHUNK, _D), jnp.float32),
            pltpu.VMEM((_LANES,), jnp.float32),
            pltpu.SemaphoreType.DMA,
            pltpu.SemaphoreType.DMA,
            pltpu.SemaphoreType.DMA,
            pltpu.SemaphoreType.DMA,
            pltpu.SemaphoreType.DMA,
        ],
    )(feat, y, w, gamma)
    return jnp.sum(out)


def kernel(output_features, y_truth, fixed_weights, centers_gamma):
    y = y_truth.astype(jnp.int32)
    gamma = centers_gamma.reshape(-1)
    return _run(output_features, y, fixed_weights, gamma)


# speedup vs baseline: 1.0292x; 1.0292x over previous
import jax
import jax.numpy as jnp
from jax import lax
from jax.experimental import pallas as pl
from jax.experimental.pallas import tpu as pltpu
from jax.experimental.pallas import tpu_sc as plsc

_B = 16384
_D = 128
_LOSS_WEIGHT = 0.005
_SCALE = 0.5 * _LOSS_WEIGHT / _B
_NC = 2
_NS = 16
_NW = _NC * _NS
_BPW = _B // _NW
_CHUNK = 128
_NBUF = 3
_CHUNKS = tuple((o, _CHUNK) for o in range(0, _BPW, _CHUNK))
_NCH = len(_CHUNKS)
_LANES = 16
_DBLK = _D // _LANES


def _center_loss_body(feat_hbm, y_hbm, w_hbm, gamma_hbm, out_hbm,
                      idx_v, gam_v, feat_v, w_v, acc_v,
                      gsem0, gsem1, gsem2, gsem3,
                      fsem0, fsem1, fsem2, wsem0, wsem1, wsem2):
    wid = lax.axis_index("s") * _NC + lax.axis_index("c")
    base = wid * _BPW

    fsems = (fsem0, fsem1, fsem2)
    wsems = (wsem0, wsem1, wsem2)
    gsems = (gsem0, gsem1, gsem2, gsem3)
    fcp = [None] * _NCH
    wcp = [None] * _NCH
    gcp = [None] * _NCH

    def fire_gamma(ch):
        off, size = _CHUNKS[ch]
        gcp[ch] = pltpu.async_copy(
            gamma_hbm.at[idx_v.at[pl.ds(off, size)]],
            gam_v.at[pl.ds(off, size)],
            gsems[ch])

    def fire(ch):
        buf = ch % _NBUF
        off, size = _CHUNKS[ch]
        fcp[ch] = pltpu.async_copy(
            feat_hbm.at[pl.ds(base + off, size)],
            feat_v.at[buf, pl.ds(0, size)], fsems[buf])
        wcp[ch] = pltpu.async_copy(
            w_hbm.at[idx_v.at[pl.ds(off, size)]],
            w_v.at[buf, pl.ds(0, size)], wsems[buf])

    fcp[0] = pltpu.async_copy(
        feat_hbm.at[pl.ds(base, _CHUNK)], feat_v.at[0], fsems[0])
    pltpu.sync_copy(y_hbm.at[pl.ds(base, _BPW)], idx_v)
    wcp[0] = pltpu.async_copy(
        w_hbm.at[idx_v.at[pl.ds(0, _CHUNK)]], w_v.at[0], wsems[0])
    fire_gamma(0)
    fire(1)
    fire_gamma(1)

    acc = jnp.zeros((_LANES,), jnp.float32)
    for ch in range(_NCH):
        if ch + 2 < _NCH:
            fire(ch + 2)
            fire_gamma(ch + 2)
        fcp[ch].wait()
        wcp[ch].wait()
        gcp[ch].wait()
        buf = ch % _NBUF
        off, size = _CHUNKS[ch]

        @plsc.parallel_loop(0, size, unroll=2, carry=acc)
        def acc(r, acc, off=off, buf=buf):
            l = jnp.bitwise_and(r, _LANES - 1)
            g16 = gam_v[pl.ds(off + (r - l), _LANES)]
            g = g16.at[jnp.full((_LANES,), l, jnp.int32)].get(
                mode="promise_in_bounds")
            for j in range(_DBLK):
                xv = feat_v[buf, r, pl.ds(j * _LANES, _LANES)]
                wv = w_v[buf, r, pl.ds(j * _LANES, _LANES)]
                d = xv - g * wv
                acc = acc + d * d
            return acc

    acc_v[...] = acc * _SCALE
    pltpu.sync_copy(acc_v, out_hbm.at[wid])


@jax.jit
def _run(feat, y, w, gamma):
    mesh = plsc.VectorSubcoreMesh(core_axis_name="c", subcore_axis_name="s")
    out = pl.kernel(
        _center_loss_body,
        mesh=mesh,
        out_type=jax.ShapeDtypeStruct((_NW, _LANES), jnp.float32),
        scratch_types=[
            pltpu.VMEM((_BPW,), jnp.int32),
            pltpu.VMEM((_BPW,), jnp.float32),
            pltpu.VMEM((_NBUF, _CHUNK, _D), jnp.float32),
            pltpu.VMEM((_NBUF, _CHUNK, _D), jnp.float32),
            pltpu.VMEM((_LANES,), jnp.float32),
            pltpu.SemaphoreType.DMA,
            pltpu.SemaphoreType.DMA,
            pltpu.SemaphoreType.DMA,
            pltpu.SemaphoreType.DMA,
            pltpu.SemaphoreType.DMA,
            pltpu.SemaphoreType.DMA,
            pltpu.SemaphoreType.DMA,
            pltpu.SemaphoreType.DMA,
            pltpu.SemaphoreType.DMA,
            pltpu.SemaphoreType.DMA,
        ],
    )(feat, y, w, gamma)
    return jnp.sum(out)


def kernel(output_features, y_truth, fixed_weights, centers_gamma):
    y = y_truth.astype(jnp.int32)
    gamma = centers_gamma.reshape(-1)
    return _run(output_features, y, fixed_weights, gamma)


# speedup vs baseline: 1.0620x; 1.0319x over previous
import jax
import jax.numpy as jnp
from jax import lax
from jax.experimental import pallas as pl
from jax.experimental.pallas import tpu as pltpu
from jax.experimental.pallas import tpu_sc as plsc

_B = 16384
_D = 128
_LOSS_WEIGHT = 0.005
_SCALE = 0.5 * _LOSS_WEIGHT / _B
_NC = 2
_NS = 16
_NW = _NC * _NS
_BPW = _B // _NW
_CHUNK = 128
_NBUF = 3
_CHUNKS = tuple((o, _CHUNK) for o in range(0, _BPW, _CHUNK))
_NCH = len(_CHUNKS)
_LANES = 16
_DBLK = _D // _LANES


def _center_loss_body(feat_hbm, y_hbm, w_hbm, gamma_hbm, out_hbm,
                      idx_v, gam_v, feat_v, w_v, acc_v,
                      gsem, fsem0, fsem1, fsem2, wsem0, wsem1, wsem2):
    wid = lax.axis_index("s") * _NC + lax.axis_index("c")
    base = wid * _BPW

    fsems = (fsem0, fsem1, fsem2)
    wsems = (wsem0, wsem1, wsem2)
    fcp = [None] * _NCH
    wcp = [None] * _NCH

    def fire(ch):
        buf = ch % _NBUF
        off, size = _CHUNKS[ch]
        fcp[ch] = pltpu.async_copy(
            feat_hbm.at[pl.ds(base + off, size)],
            feat_v.at[buf, pl.ds(0, size)], fsems[buf])
        wcp[ch] = pltpu.async_copy(
            w_hbm.at[idx_v.at[pl.ds(off, size)]],
            w_v.at[buf, pl.ds(0, size)], wsems[buf])

    fcp[0] = pltpu.async_copy(
        feat_hbm.at[pl.ds(base, _CHUNK)], feat_v.at[0], fsems[0])
    pltpu.sync_copy(y_hbm.at[pl.ds(base, _BPW)], idx_v)
    wcp[0] = pltpu.async_copy(
        w_hbm.at[idx_v.at[pl.ds(0, _CHUNK)]], w_v.at[0], wsems[0])
    gcopies = [
        pltpu.async_copy(
            gamma_hbm.at[idx_v.at[pl.ds(off, size)]],
            gam_v.at[pl.ds(off, size)],
            gsem)
        for off, size in _CHUNKS]
    fire(1)
    for cp in gcopies:
        cp.wait()

    acc = jnp.zeros((_LANES,), jnp.float32)
    for ch in range(_NCH):
        if ch + 2 < _NCH:
            fire(ch + 2)
        fcp[ch].wait()
        wcp[ch].wait()
        buf = ch % _NBUF
        off, size = _CHUNKS[ch]

        @plsc.parallel_loop(0, size, unroll=4, carry=acc)
        def acc(r, acc, off=off, buf=buf):
            l = jnp.bitwise_and(r, _LANES - 1)
            g16 = gam_v[pl.ds(off + (r - l), _LANES)]
            g = g16.at[jnp.full((_LANES,), l, jnp.int32)].get(
                mode="promise_in_bounds")
            for j in range(_DBLK):
                xv = feat_v[buf, r, pl.ds(j * _LANES, _LANES)]
                wv = w_v[buf, r, pl.ds(j * _LANES, _LANES)]
                d = xv - g * wv
                acc = acc + d * d
            return acc

    acc_v[...] = acc * _SCALE
    pltpu.sync_copy(acc_v, out_hbm.at[wid])


@jax.jit
def _run(feat, y, w, gamma):
    mesh = plsc.VectorSubcoreMesh(core_axis_name="c", subcore_axis_name="s")
    out = pl.kernel(
        _center_loss_body,
        mesh=mesh,
        out_type=jax.ShapeDtypeStruct((_NW, _LANES), jnp.float32),
        scratch_types=[
            pltpu.VMEM((_BPW,), jnp.int32),
            pltpu.VMEM((_BPW,), jnp.float32),
            pltpu.VMEM((_NBUF, _CHUNK, _D), jnp.float32),
            pltpu.VMEM((_NBUF, _CHUNK, _D), jnp.float32),
            pltpu.VMEM((_LANES,), jnp.float32),
            pltpu.SemaphoreType.DMA,
            pltpu.SemaphoreType.DMA,
            pltpu.SemaphoreType.DMA,
            pltpu.SemaphoreType.DMA,
            pltpu.SemaphoreType.DMA,
            pltpu.SemaphoreType.DMA,
            pltpu.SemaphoreType.DMA,
        ],
    )(feat, y, w, gamma)
    return jnp.sum(out)


def kernel(output_features, y_truth, fixed_weights, centers_gamma):
    y = y_truth.astype(jnp.int32)
    gamma = centers_gamma.reshape(-1)
    return _run(output_features, y, fixed_weights, gamma)


# speedup vs baseline: 1.0640x; 1.0019x over previous
import jax
import jax.numpy as jnp
from jax import lax
from jax.experimental import pallas as pl
from jax.experimental.pallas import tpu as pltpu
from jax.experimental.pallas import tpu_sc as plsc

_B = 16384
_D = 128
_LOSS_WEIGHT = 0.005
_SCALE = 0.5 * _LOSS_WEIGHT / _B
_NC = 2
_NS = 16
_NW = _NC * _NS
_BPW = _B // _NW
_CHUNK = 128
_NBUF = 3
_CHUNKS = tuple((o, _CHUNK) for o in range(0, _BPW, _CHUNK))
_NCH = len(_CHUNKS)
_LANES = 16
_DBLK = _D // _LANES


def _center_loss_body(feat_hbm, y_hbm, w_hbm, gamma_hbm, out_hbm,
                      idx_v, gam_v, feat_v, w_v, acc_v,
                      gsem, fsem0, fsem1, fsem2, wsem0, wsem1, wsem2):
    wid = lax.axis_index("c") * _NS + lax.axis_index("s")
    base = wid * _BPW

    fsems = (fsem0, fsem1, fsem2)
    wsems = (wsem0, wsem1, wsem2)
    fcp = [None] * _NCH
    wcp = [None] * _NCH

    def fire(ch):
        buf = ch % _NBUF
        off, size = _CHUNKS[ch]
        fcp[ch] = pltpu.async_copy(
            feat_hbm.at[pl.ds(base + off, size)],
            feat_v.at[buf, pl.ds(0, size)], fsems[buf])
        wcp[ch] = pltpu.async_copy(
            w_hbm.at[idx_v.at[pl.ds(off, size)]],
            w_v.at[buf, pl.ds(0, size)], wsems[buf])

    fcp[0] = pltpu.async_copy(
        feat_hbm.at[pl.ds(base, _CHUNK)], feat_v.at[0], fsems[0])
    pltpu.sync_copy(y_hbm.at[pl.ds(base, _BPW)], idx_v)
    wcp[0] = pltpu.async_copy(
        w_hbm.at[idx_v.at[pl.ds(0, _CHUNK)]], w_v.at[0], wsems[0])
    gcopies = [
        pltpu.async_copy(
            gamma_hbm.at[idx_v.at[pl.ds(off, size)]],
            gam_v.at[pl.ds(off, size)],
            gsem)
        for off, size in _CHUNKS]
    fire(1)
    for cp in gcopies:
        cp.wait()

    acc = jnp.zeros((_LANES,), jnp.float32)
    for ch in range(_NCH):
        if ch + 2 < _NCH:
            fire(ch + 2)
        fcp[ch].wait()
        wcp[ch].wait()
        buf = ch % _NBUF
        off, size = _CHUNKS[ch]

        @plsc.parallel_loop(0, size, unroll=2, carry=acc)
        def acc(r, acc, off=off, buf=buf):
            l = jnp.bitwise_and(r, _LANES - 1)
            g16 = gam_v[pl.ds(off + (r - l), _LANES)]
            g = g16.at[jnp.full((_LANES,), l, jnp.int32)].get(
                mode="promise_in_bounds")
            for j in range(_DBLK):
                xv = feat_v[buf, r, pl.ds(j * _LANES, _LANES)]
                wv = w_v[buf, r, pl.ds(j * _LANES, _LANES)]
                d = xv - g * wv
                acc = acc + d * d
            return acc

    acc_v[...] = acc * _SCALE
    pltpu.sync_copy(acc_v, out_hbm.at[wid])


@jax.jit
def _run(feat, y, w, gamma):
    mesh = plsc.VectorSubcoreMesh(core_axis_name="c", subcore_axis_name="s")
    out = pl.kernel(
        _center_loss_body,
        mesh=mesh,
        out_type=jax.ShapeDtypeStruct((_NW, _LANES), jnp.float32),
        scratch_types=[
            pltpu.VMEM((_BPW,), jnp.int32),
            pltpu.VMEM((_BPW,), jnp.float32),
            pltpu.VMEM((_NBUF, _CHUNK, _D), jnp.float32),
            pltpu.VMEM((_NBUF, _CHUNK, _D), jnp.float32),
            pltpu.VMEM((_LANES,), jnp.float32),
            pltpu.SemaphoreType.DMA,
            pltpu.SemaphoreType.DMA,
            pltpu.SemaphoreType.DMA,
            pltpu.SemaphoreType.DMA,
            pltpu.SemaphoreType.DMA,
            pltpu.SemaphoreType.DMA,
            pltpu.SemaphoreType.DMA,
        ],
    )(feat, y, w, gamma)
    return jnp.sum(out)


def kernel(output_features, y_truth, fixed_weights, centers_gamma):
    y = y_truth.astype(jnp.int32)
    gamma = centers_gamma.reshape(-1)
    return _run(output_features, y, fixed_weights, gamma)
